# Initial kernel scaffold; baseline (speedup 1.0000x reference)
import jax
import jax.numpy as jnp
from jax.experimental import pallas as pl


def kernel(x, batch, W_ih, W_hh, b_ih, b_hh):
    raise NotImplementedError("write your pallas kernel here")


# speedup vs baseline: 6.6358x; 6.6358x over previous
import functools

import jax
import jax.numpy as jnp
from jax.experimental import pallas as pl
from jax.experimental.pallas import tpu as pltpu

D = 512
B = 64
T = 3
N = 50000
BLK = 2000
NB = N // BLK
NEG = -1e30

_PREC = jax.lax.Precision.HIGHEST


def _body(batch_ref, x_ref, wq_ref, wr_ref, b_ref,
          out_ref,
          h_ref, c_ref, r_ref, m_ref, s_ref, u_ref):
    t = pl.program_id(0)
    i = pl.program_id(1)

    @pl.when(i == 0)
    def _lstm_and_init():
        first = (t == 0)
        h_prev = jnp.where(first, 0.0, h_ref[...])
        c_prev = jnp.where(first, 0.0, c_ref[...])
        r_prev = jnp.where(first, 0.0, r_ref[...])
        gates = (
            jax.lax.dot_general(h_prev, wq_ref[...], (((1,), (0,)), ((), ())),
                                precision=_PREC)
            + jax.lax.dot_general(r_prev, wr_ref[...], (((1,), (0,)), ((), ())),
                                  precision=_PREC)
            + b_ref[...]
        )
        gi = jax.nn.sigmoid(gates[:, 0 * D:1 * D])
        gf = jax.nn.sigmoid(gates[:, 1 * D:2 * D])
        gg = jnp.tanh(gates[:, 2 * D:3 * D])
        go = jax.nn.sigmoid(gates[:, 3 * D:4 * D])
        c_new = gf * c_prev + gi * gg
        h_new = go * jnp.tanh(c_new)
        h_ref[...] = h_new
        c_ref[...] = c_new
        m_ref[...] = jnp.full((B, 1), NEG, jnp.float32)
        s_ref[...] = jnp.zeros((B, 1), jnp.float32)
        u_ref[...] = jnp.zeros((B, D), jnp.float32)

    x_blk = x_ref[...]
    h = h_ref[...]
    e_t = jax.lax.dot_general(h, x_blk, (((1,), (1,)), ((), ())),
                              precision=_PREC)
    seg = batch_ref[0]
    mask = seg == jax.lax.broadcasted_iota(jnp.int32, (B, BLK), 0)
    e_m = jnp.where(mask, e_t, NEG)
    m_old = m_ref[...]
    m_new = jnp.maximum(m_old, jnp.max(e_m, axis=1, keepdims=True))
    p_t = jnp.where(mask, jnp.exp(e_t - m_new), 0.0)
    scale = jnp.exp(m_old - m_new)
    s_ref[...] = s_ref[...] * scale + jnp.sum(p_t, axis=1, keepdims=True)
    u_ref[...] = u_ref[...] * scale + jax.lax.dot_general(
        p_t, x_blk, (((1,), (0,)), ((), ())), precision=_PREC)
    m_ref[...] = m_new

    @pl.when(i == NB - 1)
    def _finalize():
        r = u_ref[...] / (s_ref[...] + 1e-16)
        r_ref[...] = r

        @pl.when(t == T - 1)
        def _write_out():
            out_ref[:, :D] = h_ref[...]
            out_ref[:, D:] = r


@functools.partial(jax.jit, static_argnames=())
def kernel(x, batch, W_ih, W_hh, b_ih, b_hh):
    batch3 = batch.astype(jnp.int32).reshape(NB, 1, BLK)
    wq = W_ih.T[:D] + W_hh.T
    wr = W_ih.T[D:]
    bias = (b_ih + b_hh).reshape(1, 4 * D)
    return pl.pallas_call(
        _body,
        grid=(T, NB),
        in_specs=[
            pl.BlockSpec((1, 1, BLK), lambda t, i: (i, 0, 0)),
            pl.BlockSpec((BLK, D), lambda t, i: (i, 0)),
            pl.BlockSpec((D, 4 * D), lambda t, i: (0, 0)),
            pl.BlockSpec((D, 4 * D), lambda t, i: (0, 0)),
            pl.BlockSpec((1, 4 * D), lambda t, i: (0, 0)),
        ],
        out_specs=pl.BlockSpec((B, 2 * D), lambda t, i: (0, 0)),
        out_shape=jax.ShapeDtypeStruct((B, 2 * D), jnp.float32),
        scratch_shapes=[
            pltpu.VMEM((B, D), jnp.float32),
            pltpu.VMEM((B, D), jnp.float32),
            pltpu.VMEM((B, D), jnp.float32),
            pltpu.VMEM((B, 1), jnp.float32),
            pltpu.VMEM((B, 1), jnp.float32),
            pltpu.VMEM((B, D), jnp.float32),
        ],
        compiler_params=pltpu.CompilerParams(
            dimension_semantics=("arbitrary", "arbitrary"),
        ),
    )(batch3, x, wq, wr, bias)


# speedup vs baseline: 11.5982x; 1.7478x over previous
import functools

import jax
import jax.numpy as jnp
from jax.experimental import pallas as pl
from jax.experimental.pallas import tpu as pltpu

D = 512
B = 64
T = 3
N = 50000
BLK = 2000
NB = N // BLK
NEG = -1e30

def _split(a):
    hi = a.astype(jnp.bfloat16)
    lo = (a - hi.astype(jnp.float32)).astype(jnp.bfloat16)
    return hi, lo


def _dot3(a, b, dims):
    ah, al = _split(a)
    bh, bl = _split(b)

    def d(u, v):
        return jax.lax.dot_general(u, v, (dims, ((), ())),
                                   preferred_element_type=jnp.float32)

    return d(ah, bh) + d(ah, bl) + d(al, bh)


def _body(batch_ref, x_ref, wq_ref, wr_ref, b_ref,
          out_ref,
          h_ref, c_ref, r_ref, m_ref, s_ref, u_ref):
    t = pl.program_id(0)
    i = pl.program_id(1)

    @pl.when(i == 0)
    def _lstm_and_init():
        first = (t == 0)
        h_prev = jnp.where(first, 0.0, h_ref[...])
        c_prev = jnp.where(first, 0.0, c_ref[...])
        r_prev = jnp.where(first, 0.0, r_ref[...])
        gates = (
            _dot3(h_prev, wq_ref[...], ((1,), (0,)))
            + _dot3(r_prev, wr_ref[...], ((1,), (0,)))
            + b_ref[...]
        )
        gi = jax.nn.sigmoid(gates[:, 0 * D:1 * D])
        gf = jax.nn.sigmoid(gates[:, 1 * D:2 * D])
        gg = jnp.tanh(gates[:, 2 * D:3 * D])
        go = jax.nn.sigmoid(gates[:, 3 * D:4 * D])
        c_new = gf * c_prev + gi * gg
        h_new = go * jnp.tanh(c_new)
        h_ref[...] = h_new
        c_ref[...] = c_new
        m_ref[...] = jnp.full((B, 1), NEG, jnp.float32)
        s_ref[...] = jnp.zeros((B, 1), jnp.float32)
        u_ref[...] = jnp.zeros((B, D), jnp.float32)

    x_blk = x_ref[...]
    xh, xl = _split(x_blk)
    h = h_ref[...]
    hh, hl = _split(h)

    def dxt(u, v):
        return jax.lax.dot_general(u, v, (((1,), (1,)), ((), ())),
                                   preferred_element_type=jnp.float32)

    e_t = dxt(hh, xh) + dxt(hh, xl) + dxt(hl, xh)
    seg = batch_ref[0]
    mask = seg == jax.lax.broadcasted_iota(jnp.int32, (B, BLK), 0)
    e_m = jnp.where(mask, e_t, NEG)
    m_old = m_ref[...]
    m_new = jnp.maximum(m_old, jnp.max(e_m, axis=1, keepdims=True))
    p_t = jnp.where(mask, jnp.exp(e_t - m_new), 0.0)
    scale = jnp.exp(m_old - m_new)
    s_ref[...] = s_ref[...] * scale + jnp.sum(p_t, axis=1, keepdims=True)
    ph, plo = _split(p_t)

    def dp(u, v):
        return jax.lax.dot_general(u, v, (((1,), (0,)), ((), ())),
                                   preferred_element_type=jnp.float32)

    u_ref[...] = (u_ref[...] * scale
                  + dp(ph, xh) + dp(ph, xl) + dp(plo, xh))
    m_ref[...] = m_new

    @pl.when(i == NB - 1)
    def _finalize():
        r = u_ref[...] / (s_ref[...] + 1e-16)
        r_ref[...] = r

        @pl.when(t == T - 1)
        def _write_out():
            out_ref[:, :D] = h_ref[...]
            out_ref[:, D:] = r


@functools.partial(jax.jit, static_argnames=())
def kernel(x, batch, W_ih, W_hh, b_ih, b_hh):
    batch3 = batch.astype(jnp.int32).reshape(NB, 1, BLK)
    wq = W_ih.T[:D] + W_hh.T
    wr = W_ih.T[D:]
    bias = (b_ih + b_hh).reshape(1, 4 * D)
    return pl.pallas_call(
        _body,
        grid=(T, NB),
        in_specs=[
            pl.BlockSpec((1, 1, BLK), lambda t, i: (i, 0, 0)),
            pl.BlockSpec((BLK, D), lambda t, i: (i, 0)),
            pl.BlockSpec((D, 4 * D), lambda t, i: (0, 0)),
            pl.BlockSpec((D, 4 * D), lambda t, i: (0, 0)),
            pl.BlockSpec((1, 4 * D), lambda t, i: (0, 0)),
        ],
        out_specs=pl.BlockSpec((B, 2 * D), lambda t, i: (0, 0)),
        out_shape=jax.ShapeDtypeStruct((B, 2 * D), jnp.float32),
        scratch_shapes=[
            pltpu.VMEM((B, D), jnp.float32),
            pltpu.VMEM((B, D), jnp.float32),
            pltpu.VMEM((B, D), jnp.float32),
            pltpu.VMEM((B, 1), jnp.float32),
            pltpu.VMEM((B, 1), jnp.float32),
            pltpu.VMEM((B, D), jnp.float32),
        ],
        compiler_params=pltpu.CompilerParams(
            dimension_semantics=("arbitrary", "arbitrary"),
        ),
    )(batch3, x, wq, wr, bias)
